# baseline (speedup 1.0000x reference)
import functools

import jax
import jax.numpy as jnp
from jax import lax
from jax.experimental import pallas as pl
from jax.experimental.pallas import tpu as pltpu

B, S, D, FF, E, R = 1, 2048, 768, 3072, 8, 32
ER = E * R
SCALING = 16.0 / 32.0
TILE = 512

_DN_T = (((1,), (1,)), ((), ()))


def _dott(a, b):
    return lax.dot_general(a, b, _DN_T, preferred_element_type=jnp.float32)


def _dot(a, b):
    return jnp.dot(a, b, preferred_element_type=jnp.float32)


def _fused_kernel(x_ref, wr_ref, rb_ref,
                  w1_any, b1_ref, w2_any, b2_ref,
                  a1_any, a1b_ref, bm1_any, bb1_ref,
                  a2_any, a2b_ref, bm2_any, bb2_ref,
                  out_ref, routing_ref, choice_ref,
                  w1_s, w2_s, a1_s, a2_s, bm1_s, bm2_s,
                  sw1, sw2, sa1, sa2, sb1, sb2):
    f32 = jnp.float32
    i = pl.program_id(0)

    def cp(src, dst, sem):
        return pltpu.make_async_copy(src, dst, sem)

    @pl.when(i == 0)
    def _start():
        cp(a1_any, a1_s, sa1).start()
        cp(bm1_any, bm1_s, sb1).start()
        cp(w1_any, w1_s, sw1).start()
        cp(a2_any, a2_s, sa2).start()
        cp(bm2_any, bm2_s, sb2).start()
        cp(w2_any, w2_s, sw2).start()

    xt = x_ref[...]

    logits = _dott(xt, wr_ref[...]) + rb_ref[...]
    routing = jax.nn.softmax(logits, axis=-1)
    routing_ref[...] = routing

    mx = jnp.max(routing, axis=-1, keepdims=True)
    eq = routing == mx
    iot = lax.broadcasted_iota(jnp.int32, routing.shape, 1)
    idx = jnp.min(jnp.where(eq, iot, E), axis=-1, keepdims=True)
    choice_ref[...] = (iot == idx).astype(f32)

    col = lax.broadcasted_iota(jnp.int32, (E, ER), 1) // R
    row = lax.broadcasted_iota(jnp.int32, (E, ER), 0)
    expand = (col == row).astype(f32)
    r_exp = _dot(routing, expand)

    @pl.when(i == 0)
    def _wait_lora1():
        cp(a1_any, a1_s, sa1).wait()
        cp(bm1_any, bm1_s, sb1).wait()

    h = _dott(xt, a1_s[...]) + a1b_ref[...]
    lora1 = _dot(h * r_exp, bm1_s[...]) + _dot(routing, bb1_ref[...])

    @pl.when(i == 0)
    def _wait_w1():
        cp(w1_any, w1_s, sw1).wait()

    orig1 = _dott(xt, w1_s[...]) + b1_ref[...]
    h1 = jax.nn.gelu(orig1 + SCALING * lora1)

    @pl.when(i == 0)
    def _wait_lora2():
        cp(a2_any, a2_s, sa2).wait()
        cp(bm2_any, bm2_s, sb2).wait()

    h2 = _dott(h1, a2_s[...]) + a2b_ref[...]
    lora2 = _dot(h2 * r_exp, bm2_s[...]) + _dot(routing, bb2_ref[...])

    @pl.when(i == 0)
    def _wait_w2():
        cp(w2_any, w2_s, sw2).wait()

    orig2 = _dott(h1, w2_s[...]) + b2_ref[...]
    out_ref[...] = orig2 + SCALING * lora2


@functools.partial(jax.jit, static_argnames=())
def kernel(x, router_W, router_b, fc1_W, fc1_b, fc2_W, fc2_b,
           down_A, down_A_b, down_B, down_B_b,
           up_A, up_A_b, up_B, up_B_b):
    f32 = jnp.float32
    xs = x.reshape(S, D)
    rb = router_b.reshape(1, E)
    b1 = fc1_b.reshape(1, FF)
    b2 = fc2_b.reshape(1, D)
    a1 = down_A.reshape(ER, D)
    a1b = down_A_b.reshape(1, ER)
    bm1 = down_B.transpose(0, 2, 1).reshape(ER, FF)
    a2 = up_A.reshape(ER, FF)
    a2b = up_A_b.reshape(1, ER)
    bm2 = up_B.transpose(0, 2, 1).reshape(ER, D)

    grid = (S // TILE,)
    full = lambda shape: pl.BlockSpec(shape, lambda i: (0,) * len(shape))
    tok = lambda w: pl.BlockSpec((TILE, w), lambda i: (i, 0))
    anyspec = pl.BlockSpec(memory_space=pl.ANY)

    out, routing, choice = pl.pallas_call(
        _fused_kernel,
        grid=grid,
        in_specs=[
            tok(D),
            full((E, D)), full((1, E)),
            anyspec, full((1, FF)), anyspec, full((1, D)),
            anyspec, full((1, ER)), anyspec, full((E, FF)),
            anyspec, full((1, ER)), anyspec, full((E, D)),
        ],
        out_specs=[tok(D), tok(E), tok(E)],
        out_shape=[
            jax.ShapeDtypeStruct((S, D), f32),
            jax.ShapeDtypeStruct((S, E), f32),
            jax.ShapeDtypeStruct((S, E), f32),
        ],
        scratch_shapes=[
            pltpu.VMEM((FF, D), f32),
            pltpu.VMEM((D, FF), f32),
            pltpu.VMEM((ER, D), f32),
            pltpu.VMEM((ER, FF), f32),
            pltpu.VMEM((ER, FF), f32),
            pltpu.VMEM((ER, D), f32),
            pltpu.SemaphoreType.DMA,
            pltpu.SemaphoreType.DMA,
            pltpu.SemaphoreType.DMA,
            pltpu.SemaphoreType.DMA,
            pltpu.SemaphoreType.DMA,
            pltpu.SemaphoreType.DMA,
        ],
    )(xs, router_W, rb, fc1_W, b1, fc2_W, b2,
      a1, a1b, bm1, down_B_b, a2, a2b, bm2, up_B_b)

    return (out.reshape(B, S, D),
            (routing.reshape(B, S, E), choice.reshape(B, S, E)))


# speedup vs baseline: 1.1249x; 1.1249x over previous
import functools

import jax
import jax.numpy as jnp
from jax import lax
from jax.experimental import pallas as pl

B, S, D, FF, E, R = 1, 2048, 768, 3072, 8, 32
ER = E * R
SCALING = 16.0 / 32.0
TILE = 512

_DN_T = (((1,), (1,)), ((), ()))


def _dott(a, b):
    return lax.dot_general(a, b, _DN_T, preferred_element_type=jnp.float32)


def _fused_kernel(x_ref, wr_ref, rb_ref,
                  w1_ref, b1_ref, w2_ref, b2_ref,
                  a1_ref, a1b_ref, bm1_ref, bb1_ref,
                  a2_ref, a2b_ref, bm2_ref, bb2_ref,
                  out_ref, routing_ref, choice_ref):
    f32 = jnp.float32
    xt = x_ref[...]

    logits = _dott(xt, wr_ref[...]) + rb_ref[...]
    routing = jax.nn.softmax(logits, axis=-1)
    routing_ref[...] = routing

    mx = jnp.max(routing, axis=-1, keepdims=True)
    eq = routing == mx
    iot = lax.broadcasted_iota(jnp.int32, routing.shape, 1)
    idx = jnp.min(jnp.where(eq, iot, E), axis=-1, keepdims=True)
    choice_ref[...] = (iot == idx).astype(f32)

    col = lax.broadcasted_iota(jnp.int32, (E, ER), 1) // R
    row = lax.broadcasted_iota(jnp.int32, (E, ER), 0)
    expand = (col == row).astype(f32)
    r_exp = jnp.dot(routing, expand, preferred_element_type=f32)

    h = _dott(xt, a1_ref[...]) + a1b_ref[...]
    lora1 = (jnp.dot(h * r_exp, bm1_ref[...], preferred_element_type=f32)
             + jnp.dot(routing, bb1_ref[...], preferred_element_type=f32))
    orig1 = _dott(xt, w1_ref[...]) + b1_ref[...]
    h1 = jax.nn.gelu(orig1 + SCALING * lora1)

    h2 = _dott(h1, a2_ref[...]) + a2b_ref[...]
    lora2 = (jnp.dot(h2 * r_exp, bm2_ref[...], preferred_element_type=f32)
             + jnp.dot(routing, bb2_ref[...], preferred_element_type=f32))
    orig2 = _dott(h1, w2_ref[...]) + b2_ref[...]
    out_ref[...] = orig2 + SCALING * lora2


@functools.partial(jax.jit, static_argnames=())
def kernel(x, router_W, router_b, fc1_W, fc1_b, fc2_W, fc2_b,
           down_A, down_A_b, down_B, down_B_b,
           up_A, up_A_b, up_B, up_B_b):
    f32 = jnp.float32
    xs = x.reshape(S, D)
    rb = router_b.reshape(1, E)
    b1 = fc1_b.reshape(1, FF)
    b2 = fc2_b.reshape(1, D)
    a1 = down_A.reshape(ER, D)
    a1b = down_A_b.reshape(1, ER)
    bm1 = down_B.transpose(0, 2, 1).reshape(ER, FF)
    a2 = up_A.reshape(ER, FF)
    a2b = up_A_b.reshape(1, ER)
    bm2 = up_B.transpose(0, 2, 1).reshape(ER, D)

    grid = (S // TILE,)
    full = lambda shape: pl.BlockSpec(shape, lambda i: (0,) * len(shape))
    tok = lambda w: pl.BlockSpec((TILE, w), lambda i: (i, 0))

    out, routing, choice = pl.pallas_call(
        _fused_kernel,
        grid=grid,
        in_specs=[
            tok(D),
            full((E, D)), full((1, E)),
            full((FF, D)), full((1, FF)), full((D, FF)), full((1, D)),
            full((ER, D)), full((1, ER)), full((ER, FF)), full((E, FF)),
            full((ER, FF)), full((1, ER)), full((ER, D)), full((E, D)),
        ],
        out_specs=[tok(D), tok(E), tok(E)],
        out_shape=[
            jax.ShapeDtypeStruct((S, D), f32),
            jax.ShapeDtypeStruct((S, E), f32),
            jax.ShapeDtypeStruct((S, E), f32),
        ],
    )(xs, router_W, rb, fc1_W, b1, fc2_W, b2,
      a1, a1b, bm1, down_B_b, a2, a2b, bm2, up_B_b)

    return (out.reshape(B, S, D),
            (routing.reshape(B, S, E), choice.reshape(B, S, E)))
